# Initial kernel scaffold; baseline (speedup 1.0000x reference)
import jax
import jax.numpy as jnp
from jax.experimental import pallas as pl


def kernel(x, edge_index, edge_attr, Wc1, bc1, gc1, bec1, Wc2, bc2, gc2, bec2, Wc3, bc3, gc3, bec3, Wf1, bf1, Wf2, bf2, Wf3, bf3, gf1, bef1, gf2, bef2):
    raise NotImplementedError("write your pallas kernel here")


# speedup vs baseline: 7.4142x; 7.4142x over previous
import functools

import jax
import jax.numpy as jnp
from jax import lax
from jax.experimental import pallas as pl
from jax.experimental.pallas import tpu as pltpu
from jax.experimental.pallas import tpu_sc as plsc

N = 50000
E = 800000
NW = 32
EPT = E // NW
DC = 32
NCH = 4
EB = 100
NB = EPT // EB
SST = 25
RPT = N // 16
_R = 2000
F = 128


def _sc_mesh():
    return plsc.VectorSubcoreMesh(core_axis_name="c", subcore_axis_name="s")


def _sc_aggregate(qc, src2, dst2):
    @functools.partial(
        pl.kernel,
        out_type=jax.ShapeDtypeStruct((2 * N, DC), jnp.float32),
        mesh=_sc_mesh(),
        compiler_params=pltpu.CompilerParams(use_tc_tiling_on_sc=False),
        scratch_types=[
            pltpu.VMEM_SHARED((N, DC), jnp.float32),
            pltpu.VMEM((SST, EB), jnp.int32),
            pltpu.VMEM((SST, EB), jnp.int32),
            pltpu.VMEM((EB, DC), jnp.float32),
            pltpu.VMEM((128, DC), jnp.float32),
            pltpu.SemaphoreType.DMA,
        ],
    )
    def k(qc_hbm, src_hbm, dst_hbm, out_hbm, acc, sidx, didx, rows, zb, sem):
        cid = lax.axis_index("c")
        sid = lax.axis_index("s")
        wid = sid * 2 + cid
        zeros = jnp.zeros((16,), jnp.float32)

        def zb_body(i, _):
            zb[i // 2, pl.ds((i % 2) * 16, 16)] = zeros
            return 0

        lax.fori_loop(0, 256, zb_body, 0)
        base = sid * RPT

        def zc_body(i, _):
            pltpu.sync_copy(zb, acc.at[pl.ds(base + i * 128, 128)])
            return 0

        lax.fori_loop(0, RPT // 128, zc_body, 0)
        rem = RPT - (RPT // 128) * 128
        if rem:
            pltpu.sync_copy(zb.at[pl.ds(0, rem)],
                            acc.at[pl.ds(base + RPT - rem, rem)])
        plsc.subcore_barrier()

        def stage_body(t, _):
            off = wid * NB + t * SST
            pltpu.sync_copy(src_hbm.at[pl.ds(off, SST)], sidx)
            pltpu.sync_copy(dst_hbm.at[pl.ds(off, SST)], didx)

            def ebody(j, _):
                pltpu.async_copy(qc_hbm.at[sidx.at[j]], rows, sem).wait()
                pltpu.sync_copy(rows, acc.at[didx.at[j]], add=True)
                return 0

            lax.fori_loop(0, SST, ebody, 0)
            return 0

        lax.fori_loop(0, NB // SST, stage_body, 0)
        plsc.subcore_barrier()
        pltpu.sync_copy(acc.at[pl.ds(base, RPT)],
                        out_hbm.at[pl.ds(cid * N + base, RPT)])

    return k(qc, src2, dst2)


def _tc_dinv(degp):
    def body(a_ref, b_ref, out_ref):
        out_ref[...] = lax.rsqrt(a_ref[:, :1] + b_ref[:, :1] + 1.0)

    return pl.pallas_call(
        body,
        grid=(N // _R,),
        in_specs=[
            pl.BlockSpec((_R, DC), lambda i: (i, 0)),
            pl.BlockSpec((_R, DC), lambda i: (i + N // _R, 0)),
        ],
        out_specs=pl.BlockSpec((_R, 1), lambda i: (i, 0)),
        out_shape=jax.ShapeDtypeStruct((N, 1), jnp.float32),
    )(degp, degp)


def _tc_pre(h, W, dinv, a=None, c=None):
    din = h.shape[1]
    pre_bn = a is not None

    def body(*refs):
        if pre_bn:
            h_ref, w_ref, a_ref, c_ref, dv_ref = refs[:5]
            outs = refs[5:]
            w_in = h_ref[...] * a_ref[...] + c_ref[...]
        else:
            h_ref, w_ref, dv_ref = refs[:3]
            outs = refs[3:]
            w_in = h_ref[...]
        y = jnp.dot(w_in, w_ref[...],
                    preferred_element_type=jnp.float32) * dv_ref[...]
        for k in range(NCH):
            outs[k][...] = y[:, k * DC:(k + 1) * DC]

    in_specs = [
        pl.BlockSpec((_R, din), lambda i: (i, 0)),
        pl.BlockSpec((din, F), lambda i: (0, 0)),
    ]
    args = [h, W]
    if pre_bn:
        in_specs += [pl.BlockSpec((1, F), lambda i: (0, 0))] * 2
        args += [a, c]
    in_specs += [pl.BlockSpec((_R, 1), lambda i: (i, 0))]
    args += [dinv]
    return pl.pallas_call(
        body,
        grid=(N // _R,),
        in_specs=in_specs,
        out_specs=[pl.BlockSpec((_R, DC), lambda i: (i, 0))] * NCH,
        out_shape=[jax.ShapeDtypeStruct((N, DC), jnp.float32)] * NCH,
    )(*args)


def _tc_post(rs, qs, dinv, bias):
    off = N // _R

    def body(ra0, rb0, ra1, rb1, ra2, rb2, ra3, rb3,
             q0, q1, q2, q3, dv_ref, b_ref, z_ref, s_ref, ss_ref):
        i = pl.program_id(0)
        ras = (ra0, ra1, ra2, ra3)
        rbs = (rb0, rb1, rb2, rb3)
        qss = (q0, q1, q2, q3)
        parts = [ras[c][...] + rbs[c][...] + qss[c][...] for c in range(NCH)]
        y = jnp.concatenate(parts, axis=1)
        z = jnp.maximum(y * dv_ref[...] + b_ref[...], 0.0)
        z_ref[...] = z

        @pl.when(i == 0)
        def _():
            s_ref[...] = jnp.zeros_like(s_ref)
            ss_ref[...] = jnp.zeros_like(ss_ref)

        s_ref[...] += jnp.sum(z, axis=0, keepdims=True)
        ss_ref[...] += jnp.sum(z * z, axis=0, keepdims=True)

    in_specs = []
    args = []
    for c in range(NCH):
        in_specs += [
            pl.BlockSpec((_R, DC), lambda i: (i, 0)),
            pl.BlockSpec((_R, DC), lambda i: (i + off, 0)),
        ]
        args += [rs[c], rs[c]]
    in_specs += [pl.BlockSpec((_R, DC), lambda i: (i, 0))] * NCH
    args += list(qs)
    in_specs += [
        pl.BlockSpec((_R, 1), lambda i: (i, 0)),
        pl.BlockSpec((1, F), lambda i: (0, 0)),
    ]
    args += [dinv, bias]
    return pl.pallas_call(
        body,
        grid=(N // _R,),
        in_specs=in_specs,
        out_specs=[
            pl.BlockSpec((_R, F), lambda i: (i, 0)),
            pl.BlockSpec((1, F), lambda i: (0, 0)),
            pl.BlockSpec((1, F), lambda i: (0, 0)),
        ],
        out_shape=[
            jax.ShapeDtypeStruct((N, F), jnp.float32),
            jax.ShapeDtypeStruct((1, F), jnp.float32),
            jax.ShapeDtypeStruct((1, F), jnp.float32),
        ],
    )(*args)


def _tc_mlp(z, W, u, a=None, c=None, relu_in=True, relu_out=False,
            stats=True, dout=F):
    pre_bn = a is not None

    def body(*refs):
        if pre_bn:
            z_ref, w_ref, u_ref, a_ref, c_ref = refs[:5]
            rest = refs[5:]
            w_in = z_ref[...] * a_ref[...] + c_ref[...]
            if relu_in:
                w_in = jnp.maximum(w_in, 0.0)
        else:
            z_ref, w_ref, u_ref = refs[:3]
            rest = refs[3:]
            w_in = z_ref[...]
        v = jnp.dot(w_in, w_ref[...],
                    preferred_element_type=jnp.float32) + u_ref[...]
        if relu_out:
            v = jnp.maximum(v, 0.0)
        if stats:
            v_ref, s_ref, ss_ref = rest
            v_ref[...] = v
            i = pl.program_id(0)

            @pl.when(i == 0)
            def _():
                s_ref[...] = jnp.zeros_like(s_ref)
                ss_ref[...] = jnp.zeros_like(ss_ref)

            s_ref[...] += jnp.sum(v, axis=0, keepdims=True)
            ss_ref[...] += jnp.sum(v * v, axis=0, keepdims=True)
        else:
            rest[0][...] = v

    in_specs = [
        pl.BlockSpec((_R, F), lambda i: (i, 0)),
        pl.BlockSpec((F, dout), lambda i: (0, 0)),
        pl.BlockSpec((1, dout), lambda i: (0, 0)),
    ]
    args = [z, W, u]
    if pre_bn:
        in_specs += [pl.BlockSpec((1, F), lambda i: (0, 0))] * 2
        args += [a, c]
    out_specs = [pl.BlockSpec((_R, dout), lambda i: (i, 0))]
    out_shape = [jax.ShapeDtypeStruct((N, dout), jnp.float32)]
    if stats:
        out_specs += [pl.BlockSpec((1, dout), lambda i: (0, 0))] * 2
        out_shape += [jax.ShapeDtypeStruct((1, dout), jnp.float32)] * 2
    res = pl.pallas_call(
        body,
        grid=(N // _R,),
        in_specs=in_specs,
        out_specs=out_specs,
        out_shape=out_shape,
    )(*args)
    return res if stats else res[0]


def _padw(W, rpad=F):
    din, dout = W.shape
    return jnp.pad(W, ((0, rpad - din), (0, F - dout)))


def _padv(v, n=F):
    return jnp.pad(v, (0, n - v.shape[0])).reshape(1, n)


def _fold(s, ss, g, be):
    mu = s[0, :100] / N
    var = ss[0, :100] / N - mu * mu
    a = g * lax.rsqrt(var + 1e-5)
    c = be - mu * a
    return a, c


def _gcn_layer(h, Wp, bias, dinv, src2, dst2, a=None, c=None):
    qs = _tc_pre(h, Wp, dinv, a=a, c=c)
    rs = [_sc_aggregate(qs[k], src2, dst2) for k in range(NCH)]
    return _tc_post(rs, qs, dinv, bias)


def kernel(x, edge_index, edge_attr, Wc1, bc1, gc1, bec1, Wc2, bc2, gc2,
           bec2, Wc3, bc3, gc3, bec3, Wf1, bf1, Wf2, bf2, Wf3, bf3, gf1,
           bef1, gf2, bef2):
    src2 = edge_index[0].reshape(E // EB, EB)
    dst1 = edge_index[1]
    dst2 = dst1.reshape(E // EB, EB)

    degp = _sc_aggregate(jnp.ones((N, DC), jnp.float32), src2, dst2)
    dinv = _tc_dinv(degp)

    z, s, ss = _gcn_layer(x, _padw(Wc1, 75), _padv(bc1), dinv, src2, dst2)
    a, c = _fold(s, ss, gc1, bec1)
    z, s, ss = _gcn_layer(z, _padw(Wc2), _padv(bc2), dinv, src2, dst2,
                          a=_padv(a), c=_padv(c))
    a, c = _fold(s, ss, gc2, bec2)
    z, s, ss = _gcn_layer(z, _padw(Wc3), _padv(bc3), dinv, src2, dst2,
                          a=_padv(a), c=_padv(c))
    a, c = _fold(s, ss, gc3, bec3)

    v4, s4, ss4 = _tc_mlp(z, _padw(Wf1), _padv(bf1),
                          a=_padv(a), c=_padv(c), relu_in=False)
    a4, c4 = _fold(s4, ss4, gf1, bef1)
    v5, s5, ss5 = _tc_mlp(v4, _padw(Wf2), _padv(bf2),
                          a=_padv(a4), c=_padv(c4))
    a5, c5 = _fold(s5, ss5, gf2, bef2)
    out = _tc_mlp(v5, _padw(Wf3), _padv(bf3, F), a=_padv(a5), c=_padv(c5),
                  relu_out=True, stats=False)
    return out[:, :3]


# speedup vs baseline: 10.7917x; 1.4555x over previous
import functools

import jax
import jax.numpy as jnp
from jax import lax
from jax.experimental import pallas as pl
from jax.experimental.pallas import tpu as pltpu
from jax.experimental.pallas import tpu_sc as plsc

N = 50000
E = 800000
NW = 32
EPT = E // NW
DC = 32
NCH = 4
EB = 100
NB = EPT // EB
SST = 25
RPT = N // 16
_R = 2000
F = 128


def _sc_mesh():
    return plsc.VectorSubcoreMesh(core_axis_name="c", subcore_axis_name="s")


def _sc_aggregate(qc, src2, dst2):
    @functools.partial(
        pl.kernel,
        out_type=jax.ShapeDtypeStruct((2 * N, DC), jnp.float32),
        mesh=_sc_mesh(),
        compiler_params=pltpu.CompilerParams(use_tc_tiling_on_sc=False),
        scratch_types=[
            pltpu.VMEM_SHARED((N, DC), jnp.float32),
            pltpu.VMEM((SST, EB), jnp.int32),
            pltpu.VMEM((SST, EB), jnp.int32),
            pltpu.VMEM((EB, DC), jnp.float32),
            pltpu.VMEM((EB, DC), jnp.float32),
            pltpu.VMEM((128, DC), jnp.float32),
            pltpu.SemaphoreType.DMA,
            pltpu.SemaphoreType.DMA,
        ],
    )
    def k(qc_hbm, src_hbm, dst_hbm, out_hbm, acc, sidx, didx, rows0, rows1,
          zb, sem0, sem1):
        cid = lax.axis_index("c")
        sid = lax.axis_index("s")
        wid = sid * 2 + cid
        zeros = jnp.zeros((16,), jnp.float32)

        def zb_body(i, _):
            zb[i // 2, pl.ds((i % 2) * 16, 16)] = zeros
            return 0

        lax.fori_loop(0, 256, zb_body, 0)
        base = sid * RPT

        def zc_body(i, _):
            pltpu.sync_copy(zb, acc.at[pl.ds(base + i * 128, 128)])
            return 0

        lax.fori_loop(0, RPT // 128, zc_body, 0)
        rem = RPT - (RPT // 128) * 128
        if rem:
            pltpu.sync_copy(zb.at[pl.ds(0, rem)],
                            acc.at[pl.ds(base + RPT - rem, rem)])
        plsc.subcore_barrier()

        def stage_body(t, _):
            off = wid * NB + t * SST
            pltpu.sync_copy(src_hbm.at[pl.ds(off, SST)], sidx)
            pltpu.sync_copy(dst_hbm.at[pl.ds(off, SST)], didx)
            pltpu.async_copy(qc_hbm.at[sidx.at[0]], rows0, sem0)
            pltpu.async_copy(qc_hbm.at[sidx.at[1]], rows1, sem1)

            def consume(rb, sb, j):
                pltpu.make_async_copy(qc_hbm.at[pl.ds(0, EB)], rb, sb).wait()
                pltpu.sync_copy(rb, acc.at[didx.at[j]], add=True)

                @pl.when(j + 2 < SST)
                def _():
                    pltpu.async_copy(qc_hbm.at[sidx.at[j + 2]], rb, sb)

            def ebody(j, _):
                lax.cond(j % 2 == 0,
                         lambda: consume(rows0, sem0, j),
                         lambda: consume(rows1, sem1, j))
                return 0

            lax.fori_loop(0, SST, ebody, 0)
            return 0

        lax.fori_loop(0, NB // SST, stage_body, 0)
        plsc.subcore_barrier()
        pltpu.sync_copy(acc.at[pl.ds(base, RPT)],
                        out_hbm.at[pl.ds(cid * N + base, RPT)])

    return k(qc, src2, dst2)


def _tc_dinv(degp):
    def body(a_ref, b_ref, out_ref):
        out_ref[...] = lax.rsqrt(a_ref[:, :1] + b_ref[:, :1] + 1.0)

    return pl.pallas_call(
        body,
        grid=(N // _R,),
        in_specs=[
            pl.BlockSpec((_R, DC), lambda i: (i, 0)),
            pl.BlockSpec((_R, DC), lambda i: (i + N // _R, 0)),
        ],
        out_specs=pl.BlockSpec((_R, 1), lambda i: (i, 0)),
        out_shape=jax.ShapeDtypeStruct((N, 1), jnp.float32),
    )(degp, degp)


def _tc_pre(h, W, dinv, a=None, c=None):
    din = h.shape[1]
    pre_bn = a is not None

    def body(*refs):
        if pre_bn:
            h_ref, w_ref, a_ref, c_ref, dv_ref = refs[:5]
            outs = refs[5:]
            w_in = h_ref[...] * a_ref[...] + c_ref[...]
        else:
            h_ref, w_ref, dv_ref = refs[:3]
            outs = refs[3:]
            w_in = h_ref[...]
        y = jnp.dot(w_in, w_ref[...],
                    preferred_element_type=jnp.float32) * dv_ref[...]
        for k in range(NCH):
            outs[k][...] = y[:, k * DC:(k + 1) * DC]

    in_specs = [
        pl.BlockSpec((_R, din), lambda i: (i, 0)),
        pl.BlockSpec((din, F), lambda i: (0, 0)),
    ]
    args = [h, W]
    if pre_bn:
        in_specs += [pl.BlockSpec((1, F), lambda i: (0, 0))] * 2
        args += [a, c]
    in_specs += [pl.BlockSpec((_R, 1), lambda i: (i, 0))]
    args += [dinv]
    return pl.pallas_call(
        body,
        grid=(N // _R,),
        in_specs=in_specs,
        out_specs=[pl.BlockSpec((_R, DC), lambda i: (i, 0))] * NCH,
        out_shape=[jax.ShapeDtypeStruct((N, DC), jnp.float32)] * NCH,
    )(*args)


def _tc_post(rs, qs, dinv, bias):
    off = N // _R

    def body(ra0, rb0, ra1, rb1, ra2, rb2, ra3, rb3,
             q0, q1, q2, q3, dv_ref, b_ref, z_ref, s_ref, ss_ref):
        i = pl.program_id(0)
        ras = (ra0, ra1, ra2, ra3)
        rbs = (rb0, rb1, rb2, rb3)
        qss = (q0, q1, q2, q3)
        parts = [ras[c][...] + rbs[c][...] + qss[c][...] for c in range(NCH)]
        y = jnp.concatenate(parts, axis=1)
        z = jnp.maximum(y * dv_ref[...] + b_ref[...], 0.0)
        z_ref[...] = z

        @pl.when(i == 0)
        def _():
            s_ref[...] = jnp.zeros_like(s_ref)
            ss_ref[...] = jnp.zeros_like(ss_ref)

        s_ref[...] += jnp.sum(z, axis=0, keepdims=True)
        ss_ref[...] += jnp.sum(z * z, axis=0, keepdims=True)

    in_specs = []
    args = []
    for c in range(NCH):
        in_specs += [
            pl.BlockSpec((_R, DC), lambda i: (i, 0)),
            pl.BlockSpec((_R, DC), lambda i: (i + off, 0)),
        ]
        args += [rs[c], rs[c]]
    in_specs += [pl.BlockSpec((_R, DC), lambda i: (i, 0))] * NCH
    args += list(qs)
    in_specs += [
        pl.BlockSpec((_R, 1), lambda i: (i, 0)),
        pl.BlockSpec((1, F), lambda i: (0, 0)),
    ]
    args += [dinv, bias]
    return pl.pallas_call(
        body,
        grid=(N // _R,),
        in_specs=in_specs,
        out_specs=[
            pl.BlockSpec((_R, F), lambda i: (i, 0)),
            pl.BlockSpec((1, F), lambda i: (0, 0)),
            pl.BlockSpec((1, F), lambda i: (0, 0)),
        ],
        out_shape=[
            jax.ShapeDtypeStruct((N, F), jnp.float32),
            jax.ShapeDtypeStruct((1, F), jnp.float32),
            jax.ShapeDtypeStruct((1, F), jnp.float32),
        ],
    )(*args)


def _tc_mlp(z, W, u, a=None, c=None, relu_in=True, relu_out=False,
            stats=True, dout=F):
    pre_bn = a is not None

    def body(*refs):
        if pre_bn:
            z_ref, w_ref, u_ref, a_ref, c_ref = refs[:5]
            rest = refs[5:]
            w_in = z_ref[...] * a_ref[...] + c_ref[...]
            if relu_in:
                w_in = jnp.maximum(w_in, 0.0)
        else:
            z_ref, w_ref, u_ref = refs[:3]
            rest = refs[3:]
            w_in = z_ref[...]
        v = jnp.dot(w_in, w_ref[...],
                    preferred_element_type=jnp.float32) + u_ref[...]
        if relu_out:
            v = jnp.maximum(v, 0.0)
        if stats:
            v_ref, s_ref, ss_ref = rest
            v_ref[...] = v
            i = pl.program_id(0)

            @pl.when(i == 0)
            def _():
                s_ref[...] = jnp.zeros_like(s_ref)
                ss_ref[...] = jnp.zeros_like(ss_ref)

            s_ref[...] += jnp.sum(v, axis=0, keepdims=True)
            ss_ref[...] += jnp.sum(v * v, axis=0, keepdims=True)
        else:
            rest[0][...] = v

    in_specs = [
        pl.BlockSpec((_R, F), lambda i: (i, 0)),
        pl.BlockSpec((F, dout), lambda i: (0, 0)),
        pl.BlockSpec((1, dout), lambda i: (0, 0)),
    ]
    args = [z, W, u]
    if pre_bn:
        in_specs += [pl.BlockSpec((1, F), lambda i: (0, 0))] * 2
        args += [a, c]
    out_specs = [pl.BlockSpec((_R, dout), lambda i: (i, 0))]
    out_shape = [jax.ShapeDtypeStruct((N, dout), jnp.float32)]
    if stats:
        out_specs += [pl.BlockSpec((1, dout), lambda i: (0, 0))] * 2
        out_shape += [jax.ShapeDtypeStruct((1, dout), jnp.float32)] * 2
    res = pl.pallas_call(
        body,
        grid=(N // _R,),
        in_specs=in_specs,
        out_specs=out_specs,
        out_shape=out_shape,
    )(*args)
    return res if stats else res[0]


def _padw(W, rpad=F):
    din, dout = W.shape
    return jnp.pad(W, ((0, rpad - din), (0, F - dout)))


def _padv(v, n=F):
    return jnp.pad(v, (0, n - v.shape[0])).reshape(1, n)


def _fold(s, ss, g, be):
    mu = s[0, :100] / N
    var = ss[0, :100] / N - mu * mu
    a = g * lax.rsqrt(var + 1e-5)
    c = be - mu * a
    return a, c


def _gcn_layer(h, Wp, bias, dinv, src2, dst2, a=None, c=None):
    qs = _tc_pre(h, Wp, dinv, a=a, c=c)
    rs = [_sc_aggregate(qs[k], src2, dst2) for k in range(NCH)]
    return _tc_post(rs, qs, dinv, bias)


def kernel(x, edge_index, edge_attr, Wc1, bc1, gc1, bec1, Wc2, bc2, gc2,
           bec2, Wc3, bc3, gc3, bec3, Wf1, bf1, Wf2, bf2, Wf3, bf3, gf1,
           bef1, gf2, bef2):
    src2 = edge_index[0].reshape(E // EB, EB)
    dst1 = edge_index[1]
    dst2 = dst1.reshape(E // EB, EB)

    degp = _sc_aggregate(jnp.ones((N, DC), jnp.float32), src2, dst2)
    dinv = _tc_dinv(degp)

    z, s, ss = _gcn_layer(x, _padw(Wc1, 75), _padv(bc1), dinv, src2, dst2)
    a, c = _fold(s, ss, gc1, bec1)
    z, s, ss = _gcn_layer(z, _padw(Wc2), _padv(bc2), dinv, src2, dst2,
                          a=_padv(a), c=_padv(c))
    a, c = _fold(s, ss, gc2, bec2)
    z, s, ss = _gcn_layer(z, _padw(Wc3), _padv(bc3), dinv, src2, dst2,
                          a=_padv(a), c=_padv(c))
    a, c = _fold(s, ss, gc3, bec3)

    v4, s4, ss4 = _tc_mlp(z, _padw(Wf1), _padv(bf1),
                          a=_padv(a), c=_padv(c), relu_in=False)
    a4, c4 = _fold(s4, ss4, gf1, bef1)
    v5, s5, ss5 = _tc_mlp(v4, _padw(Wf2), _padv(bf2),
                          a=_padv(a4), c=_padv(c4))
    a5, c5 = _fold(s5, ss5, gf2, bef2)
    out = _tc_mlp(v5, _padw(Wf3), _padv(bf3, F), a=_padv(a5), c=_padv(c5),
                  relu_out=True, stats=False)
    return out[:, :3]


# speedup vs baseline: 11.7028x; 1.0844x over previous
import functools

import jax
import jax.numpy as jnp
from jax import lax
from jax.experimental import pallas as pl
from jax.experimental.pallas import tpu as pltpu
from jax.experimental.pallas import tpu_sc as plsc

N = 50000
E = 800000
NW = 32
EPT = E // NW
DC = 32
NCH = 4
EB = 100
NB = EPT // EB
SST = 25
RPT = N // 16
_R = 2000
F = 128


def _sc_mesh():
    return plsc.VectorSubcoreMesh(core_axis_name="c", subcore_axis_name="s")


def _sc_aggregate(qc, src2, dst2):
    @functools.partial(
        pl.kernel,
        out_type=jax.ShapeDtypeStruct((2 * N, DC), jnp.float32),
        mesh=_sc_mesh(),
        compiler_params=pltpu.CompilerParams(use_tc_tiling_on_sc=False),
        scratch_types=[
            pltpu.VMEM_SHARED((N, DC), jnp.float32),
            pltpu.VMEM((SST, EB), jnp.int32),
            pltpu.VMEM((SST, EB), jnp.int32),
            pltpu.VMEM((EB, DC), jnp.float32),
            pltpu.VMEM((EB, DC), jnp.float32),
            pltpu.VMEM((EB, DC), jnp.float32),
            pltpu.VMEM((EB, DC), jnp.float32),
            pltpu.VMEM((64, DC), jnp.float32),
            pltpu.SemaphoreType.DMA,
            pltpu.SemaphoreType.DMA,
            pltpu.SemaphoreType.DMA,
            pltpu.SemaphoreType.DMA,
            pltpu.SemaphoreType.DMA,
            pltpu.SemaphoreType.DMA,
            pltpu.SemaphoreType.DMA,
            pltpu.SemaphoreType.DMA,
        ],
    )
    def k(qc_hbm, src_hbm, dst_hbm, out_hbm, acc, sidx, didx, r0, r1, r2, r3,
          zb, g0, g1, g2, g3, s0, s1, s2, s3):
        cid = lax.axis_index("c")
        sid = lax.axis_index("s")
        wid = sid * 2 + cid
        zeros = jnp.zeros((16,), jnp.float32)

        def zb_body(i, _):
            zb[i // 2, pl.ds((i % 2) * 16, 16)] = zeros
            return 0

        lax.fori_loop(0, 128, zb_body, 0)
        base = sid * RPT

        def zc_body(i, _):
            pltpu.sync_copy(zb, acc.at[pl.ds(base + i * 64, 64)])
            return 0

        lax.fori_loop(0, RPT // 64, zc_body, 0)
        rem = RPT - (RPT // 64) * 64
        if rem:
            pltpu.sync_copy(zb.at[pl.ds(0, rem)],
                            acc.at[pl.ds(base + RPT - rem, rem)])
        plsc.subcore_barrier()

        rbufs = (r0, r1, r2, r3)
        gsems = (g0, g1, g2, g3)
        ssems = (s0, s1, s2, s3)

        def wait_g(b):
            pltpu.make_async_copy(qc_hbm.at[pl.ds(0, EB)], rbufs[b],
                                  gsems[b]).wait()

        def wait_s(b):
            pltpu.make_async_copy(rbufs[b], acc.at[pl.ds(0, EB)],
                                  ssems[b]).wait()

        def stage_body(t, _):
            off = wid * NB + t * SST
            pltpu.sync_copy(src_hbm.at[pl.ds(off, SST)], sidx)
            pltpu.sync_copy(dst_hbm.at[pl.ds(off, SST)], didx)
            pltpu.async_copy(qc_hbm.at[sidx.at[0]], r0, g0)
            pltpu.async_copy(qc_hbm.at[sidx.at[1]], r1, g1)

            def consume(b, j):
                wait_g(b)
                pltpu.async_copy(rbufs[b], acc.at[didx.at[j]], ssems[b],
                                 add=True)
                b2 = (b + 2) % 4

                @pl.when(j >= 2)
                def _():
                    wait_s(b2)

                @pl.when(j + 2 < SST)
                def _():
                    pltpu.async_copy(qc_hbm.at[sidx.at[j + 2]], rbufs[b2],
                                     gsems[b2])

            def ebody(j, _):
                m = j % 4
                lax.cond(
                    m < 2,
                    lambda: lax.cond(m == 0, lambda: consume(0, j),
                                     lambda: consume(1, j)),
                    lambda: lax.cond(m == 2, lambda: consume(2, j),
                                     lambda: consume(3, j)))
                return 0

            lax.fori_loop(0, SST, ebody, 0)
            wait_s((SST - 2) % 4)
            wait_s((SST - 1) % 4)
            return 0

        lax.fori_loop(0, NB // SST, stage_body, 0)
        plsc.subcore_barrier()
        pltpu.sync_copy(acc.at[pl.ds(base, RPT)],
                        out_hbm.at[pl.ds(cid * N + base, RPT)])

    return k(qc, src2, dst2)


def _tc_dinv(degp):
    def body(a_ref, b_ref, out_ref):
        out_ref[...] = lax.rsqrt(a_ref[:, :1] + b_ref[:, :1] + 1.0)

    return pl.pallas_call(
        body,
        grid=(N // _R,),
        in_specs=[
            pl.BlockSpec((_R, DC), lambda i: (i, 0)),
            pl.BlockSpec((_R, DC), lambda i: (i + N // _R, 0)),
        ],
        out_specs=pl.BlockSpec((_R, 1), lambda i: (i, 0)),
        out_shape=jax.ShapeDtypeStruct((N, 1), jnp.float32),
    )(degp, degp)


def _tc_pre(h, W, dinv, a=None, c=None):
    din = h.shape[1]
    pre_bn = a is not None

    def body(*refs):
        if pre_bn:
            h_ref, w_ref, a_ref, c_ref, dv_ref = refs[:5]
            outs = refs[5:]
            w_in = h_ref[...] * a_ref[...] + c_ref[...]
        else:
            h_ref, w_ref, dv_ref = refs[:3]
            outs = refs[3:]
            w_in = h_ref[...]
        y = jnp.dot(w_in, w_ref[...],
                    preferred_element_type=jnp.float32) * dv_ref[...]
        for k in range(NCH):
            outs[k][...] = y[:, k * DC:(k + 1) * DC]

    in_specs = [
        pl.BlockSpec((_R, din), lambda i: (i, 0)),
        pl.BlockSpec((din, F), lambda i: (0, 0)),
    ]
    args = [h, W]
    if pre_bn:
        in_specs += [pl.BlockSpec((1, F), lambda i: (0, 0))] * 2
        args += [a, c]
    in_specs += [pl.BlockSpec((_R, 1), lambda i: (i, 0))]
    args += [dinv]
    return pl.pallas_call(
        body,
        grid=(N // _R,),
        in_specs=in_specs,
        out_specs=[pl.BlockSpec((_R, DC), lambda i: (i, 0))] * NCH,
        out_shape=[jax.ShapeDtypeStruct((N, DC), jnp.float32)] * NCH,
    )(*args)


def _tc_post(rs, qs, dinv, bias):
    off = N // _R

    def body(ra0, rb0, ra1, rb1, ra2, rb2, ra3, rb3,
             q0, q1, q2, q3, dv_ref, b_ref, z_ref, s_ref, ss_ref):
        i = pl.program_id(0)
        ras = (ra0, ra1, ra2, ra3)
        rbs = (rb0, rb1, rb2, rb3)
        qss = (q0, q1, q2, q3)
        parts = [ras[c][...] + rbs[c][...] + qss[c][...] for c in range(NCH)]
        y = jnp.concatenate(parts, axis=1)
        z = jnp.maximum(y * dv_ref[...] + b_ref[...], 0.0)
        z_ref[...] = z

        @pl.when(i == 0)
        def _():
            s_ref[...] = jnp.zeros_like(s_ref)
            ss_ref[...] = jnp.zeros_like(ss_ref)

        s_ref[...] += jnp.sum(z, axis=0, keepdims=True)
        ss_ref[...] += jnp.sum(z * z, axis=0, keepdims=True)

    in_specs = []
    args = []
    for c in range(NCH):
        in_specs += [
            pl.BlockSpec((_R, DC), lambda i: (i, 0)),
            pl.BlockSpec((_R, DC), lambda i: (i + off, 0)),
        ]
        args += [rs[c], rs[c]]
    in_specs += [pl.BlockSpec((_R, DC), lambda i: (i, 0))] * NCH
    args += list(qs)
    in_specs += [
        pl.BlockSpec((_R, 1), lambda i: (i, 0)),
        pl.BlockSpec((1, F), lambda i: (0, 0)),
    ]
    args += [dinv, bias]
    return pl.pallas_call(
        body,
        grid=(N // _R,),
        in_specs=in_specs,
        out_specs=[
            pl.BlockSpec((_R, F), lambda i: (i, 0)),
            pl.BlockSpec((1, F), lambda i: (0, 0)),
            pl.BlockSpec((1, F), lambda i: (0, 0)),
        ],
        out_shape=[
            jax.ShapeDtypeStruct((N, F), jnp.float32),
            jax.ShapeDtypeStruct((1, F), jnp.float32),
            jax.ShapeDtypeStruct((1, F), jnp.float32),
        ],
    )(*args)


def _tc_mlp(z, W, u, a=None, c=None, relu_in=True, relu_out=False,
            stats=True, dout=F):
    pre_bn = a is not None

    def body(*refs):
        if pre_bn:
            z_ref, w_ref, u_ref, a_ref, c_ref = refs[:5]
            rest = refs[5:]
            w_in = z_ref[...] * a_ref[...] + c_ref[...]
            if relu_in:
                w_in = jnp.maximum(w_in, 0.0)
        else:
            z_ref, w_ref, u_ref = refs[:3]
            rest = refs[3:]
            w_in = z_ref[...]
        v = jnp.dot(w_in, w_ref[...],
                    preferred_element_type=jnp.float32) + u_ref[...]
        if relu_out:
            v = jnp.maximum(v, 0.0)
        if stats:
            v_ref, s_ref, ss_ref = rest
            v_ref[...] = v
            i = pl.program_id(0)

            @pl.when(i == 0)
            def _():
                s_ref[...] = jnp.zeros_like(s_ref)
                ss_ref[...] = jnp.zeros_like(ss_ref)

            s_ref[...] += jnp.sum(v, axis=0, keepdims=True)
            ss_ref[...] += jnp.sum(v * v, axis=0, keepdims=True)
        else:
            rest[0][...] = v

    in_specs = [
        pl.BlockSpec((_R, F), lambda i: (i, 0)),
        pl.BlockSpec((F, dout), lambda i: (0, 0)),
        pl.BlockSpec((1, dout), lambda i: (0, 0)),
    ]
    args = [z, W, u]
    if pre_bn:
        in_specs += [pl.BlockSpec((1, F), lambda i: (0, 0))] * 2
        args += [a, c]
    out_specs = [pl.BlockSpec((_R, dout), lambda i: (i, 0))]
    out_shape = [jax.ShapeDtypeStruct((N, dout), jnp.float32)]
    if stats:
        out_specs += [pl.BlockSpec((1, dout), lambda i: (0, 0))] * 2
        out_shape += [jax.ShapeDtypeStruct((1, dout), jnp.float32)] * 2
    res = pl.pallas_call(
        body,
        grid=(N // _R,),
        in_specs=in_specs,
        out_specs=out_specs,
        out_shape=out_shape,
    )(*args)
    return res if stats else res[0]


def _padw(W, rpad=F):
    din, dout = W.shape
    return jnp.pad(W, ((0, rpad - din), (0, F - dout)))


def _padv(v, n=F):
    return jnp.pad(v, (0, n - v.shape[0])).reshape(1, n)


def _fold(s, ss, g, be):
    mu = s[0, :100] / N
    var = ss[0, :100] / N - mu * mu
    a = g * lax.rsqrt(var + 1e-5)
    c = be - mu * a
    return a, c


def _gcn_layer(h, Wp, bias, dinv, src2, dst2, a=None, c=None):
    qs = _tc_pre(h, Wp, dinv, a=a, c=c)
    rs = [_sc_aggregate(qs[k], src2, dst2) for k in range(NCH)]
    return _tc_post(rs, qs, dinv, bias)


def kernel(x, edge_index, edge_attr, Wc1, bc1, gc1, bec1, Wc2, bc2, gc2,
           bec2, Wc3, bc3, gc3, bec3, Wf1, bf1, Wf2, bf2, Wf3, bf3, gf1,
           bef1, gf2, bef2):
    src2 = edge_index[0].reshape(E // EB, EB)
    dst1 = edge_index[1]
    dst2 = dst1.reshape(E // EB, EB)

    degp = _sc_aggregate(jnp.ones((N, DC), jnp.float32), src2, dst2)
    dinv = _tc_dinv(degp)

    z, s, ss = _gcn_layer(x, _padw(Wc1, 75), _padv(bc1), dinv, src2, dst2)
    a, c = _fold(s, ss, gc1, bec1)
    z, s, ss = _gcn_layer(z, _padw(Wc2), _padv(bc2), dinv, src2, dst2,
                          a=_padv(a), c=_padv(c))
    a, c = _fold(s, ss, gc2, bec2)
    z, s, ss = _gcn_layer(z, _padw(Wc3), _padv(bc3), dinv, src2, dst2,
                          a=_padv(a), c=_padv(c))
    a, c = _fold(s, ss, gc3, bec3)

    v4, s4, ss4 = _tc_mlp(z, _padw(Wf1), _padv(bf1),
                          a=_padv(a), c=_padv(c), relu_in=False)
    a4, c4 = _fold(s4, ss4, gf1, bef1)
    v5, s5, ss5 = _tc_mlp(v4, _padw(Wf2), _padv(bf2),
                          a=_padv(a4), c=_padv(c4))
    a5, c5 = _fold(s5, ss5, gf2, bef2)
    out = _tc_mlp(v5, _padw(Wf3), _padv(bf3, F), a=_padv(a5), c=_padv(c5),
                  relu_out=True, stats=False)
    return out[:, :3]


# speedup vs baseline: 12.0054x; 1.0259x over previous
import functools

import jax
import jax.numpy as jnp
from jax import lax
from jax.experimental import pallas as pl
from jax.experimental.pallas import tpu as pltpu
from jax.experimental.pallas import tpu_sc as plsc

N = 50000
E = 800000
NW = 32
EPT = E // NW
DC = 32
NCH = 4
EB = 100
NB = EPT // EB
SST = 25
RPT = N // 16
_R = 2000
F = 128


def _sc_mesh():
    return plsc.VectorSubcoreMesh(core_axis_name="c", subcore_axis_name="s")


def _sc_aggregate(qc, src2, dst2, const_rows=False):
    @functools.partial(
        pl.kernel,
        out_type=jax.ShapeDtypeStruct((2 * N, DC), jnp.float32),
        mesh=_sc_mesh(),
        compiler_params=pltpu.CompilerParams(use_tc_tiling_on_sc=False),
        scratch_types=[
            pltpu.VMEM_SHARED((N, DC), jnp.float32),
            pltpu.VMEM((SST, EB), jnp.int32),
            pltpu.VMEM((SST, EB), jnp.int32),
            pltpu.VMEM((EB, DC), jnp.float32),
            pltpu.VMEM((EB, DC), jnp.float32),
            pltpu.VMEM((EB, DC), jnp.float32),
            pltpu.VMEM((EB, DC), jnp.float32),
            pltpu.VMEM((64, DC), jnp.float32),
            pltpu.SemaphoreType.DMA,
            pltpu.SemaphoreType.DMA,
            pltpu.SemaphoreType.DMA,
            pltpu.SemaphoreType.DMA,
            pltpu.SemaphoreType.DMA,
            pltpu.SemaphoreType.DMA,
            pltpu.SemaphoreType.DMA,
            pltpu.SemaphoreType.DMA,
        ],
    )
    def k(qc_hbm, src_hbm, dst_hbm, out_hbm, acc, sidx, didx, r0, r1, r2, r3,
          zb, g0, g1, g2, g3, s0, s1, s2, s3):
        cid = lax.axis_index("c")
        sid = lax.axis_index("s")
        wid = sid * 2 + cid
        zeros = jnp.zeros((16,), jnp.float32)

        def zb_body(i, _):
            zb[i // 2, pl.ds((i % 2) * 16, 16)] = zeros
            return 0

        lax.fori_loop(0, 128, zb_body, 0)
        base = sid * RPT

        def zc_body(i, _):
            pltpu.sync_copy(zb, acc.at[pl.ds(base + i * 64, 64)])
            return 0

        lax.fori_loop(0, RPT // 64, zc_body, 0)
        rem = RPT - (RPT // 64) * 64
        if rem:
            pltpu.sync_copy(zb.at[pl.ds(0, rem)],
                            acc.at[pl.ds(base + RPT - rem, rem)])
        plsc.subcore_barrier()

        rbufs = (r0, r1, r2, r3)
        gsems = (g0, g1, g2, g3)
        ssems = (s0, s1, s2, s3)

        def wait_g(b):
            pltpu.make_async_copy(qc_hbm.at[pl.ds(0, EB)], rbufs[b],
                                  gsems[b]).wait()

        def wait_s(b):
            pltpu.make_async_copy(rbufs[b], acc.at[pl.ds(0, EB)],
                                  ssems[b]).wait()

        if const_rows:
            for rb in rbufs:
                pltpu.sync_copy(qc_hbm.at[pl.ds(0, EB)], rb)

        def stage_body(t, _):
            off = wid * NB + t * SST
            if not const_rows:
                pltpu.sync_copy(src_hbm.at[pl.ds(off, SST)], sidx)
            pltpu.sync_copy(dst_hbm.at[pl.ds(off, SST)], didx)
            if not const_rows:
                pltpu.async_copy(qc_hbm.at[sidx.at[0]], r0, g0)
                pltpu.async_copy(qc_hbm.at[sidx.at[1]], r1, g1)

            def consume(b, j):
                if not const_rows:
                    wait_g(b)
                pltpu.async_copy(rbufs[b], acc.at[didx.at[j]], ssems[b],
                                 add=True)
                b2 = (b + 2) % 4

                @pl.when(j >= 2)
                def _():
                    wait_s(b2)

                if not const_rows:
                    @pl.when(j + 2 < SST)
                    def _():
                        pltpu.async_copy(qc_hbm.at[sidx.at[j + 2]], rbufs[b2],
                                         gsems[b2])

            def ebody(j, _):
                m = j % 4
                lax.cond(
                    m < 2,
                    lambda: lax.cond(m == 0, lambda: consume(0, j),
                                     lambda: consume(1, j)),
                    lambda: lax.cond(m == 2, lambda: consume(2, j),
                                     lambda: consume(3, j)))
                return 0

            lax.fori_loop(0, SST, ebody, 0)
            wait_s((SST - 2) % 4)
            wait_s((SST - 1) % 4)
            return 0

        lax.fori_loop(0, NB // SST, stage_body, 0)
        plsc.subcore_barrier()
        pltpu.sync_copy(acc.at[pl.ds(base, RPT)],
                        out_hbm.at[pl.ds(cid * N + base, RPT)])

    return k(qc, src2, dst2)


def _tc_dinv(degp):
    def body(a_ref, b_ref, out_ref):
        out_ref[...] = lax.rsqrt(a_ref[:, :1] + b_ref[:, :1] + 1.0)

    return pl.pallas_call(
        body,
        grid=(N // _R,),
        in_specs=[
            pl.BlockSpec((_R, DC), lambda i: (i, 0)),
            pl.BlockSpec((_R, DC), lambda i: (i + N // _R, 0)),
        ],
        out_specs=pl.BlockSpec((_R, 1), lambda i: (i, 0)),
        out_shape=jax.ShapeDtypeStruct((N, 1), jnp.float32),
    )(degp, degp)


def _tc_pre(h, W, dinv, a=None, c=None):
    din = h.shape[1]
    pre_bn = a is not None

    def body(*refs):
        if pre_bn:
            h_ref, w_ref, a_ref, c_ref, dv_ref = refs[:5]
            outs = refs[5:]
            w_in = h_ref[...] * a_ref[...] + c_ref[...]
        else:
            h_ref, w_ref, dv_ref = refs[:3]
            outs = refs[3:]
            w_in = h_ref[...]
        y = jnp.dot(w_in, w_ref[...],
                    preferred_element_type=jnp.float32) * dv_ref[...]
        for k in range(NCH):
            outs[k][...] = y[:, k * DC:(k + 1) * DC]

    in_specs = [
        pl.BlockSpec((_R, din), lambda i: (i, 0)),
        pl.BlockSpec((din, F), lambda i: (0, 0)),
    ]
    args = [h, W]
    if pre_bn:
        in_specs += [pl.BlockSpec((1, F), lambda i: (0, 0))] * 2
        args += [a, c]
    in_specs += [pl.BlockSpec((_R, 1), lambda i: (i, 0))]
    args += [dinv]
    return pl.pallas_call(
        body,
        grid=(N // _R,),
        in_specs=in_specs,
        out_specs=[pl.BlockSpec((_R, DC), lambda i: (i, 0))] * NCH,
        out_shape=[jax.ShapeDtypeStruct((N, DC), jnp.float32)] * NCH,
    )(*args)


def _tc_post(rs, qs, dinv, bias):
    off = N // _R

    def body(ra0, rb0, ra1, rb1, ra2, rb2, ra3, rb3,
             q0, q1, q2, q3, dv_ref, b_ref, z_ref, s_ref, ss_ref):
        i = pl.program_id(0)
        ras = (ra0, ra1, ra2, ra3)
        rbs = (rb0, rb1, rb2, rb3)
        qss = (q0, q1, q2, q3)
        parts = [ras[c][...] + rbs[c][...] + qss[c][...] for c in range(NCH)]
        y = jnp.concatenate(parts, axis=1)
        z = jnp.maximum(y * dv_ref[...] + b_ref[...], 0.0)
        z_ref[...] = z

        @pl.when(i == 0)
        def _():
            s_ref[...] = jnp.zeros_like(s_ref)
            ss_ref[...] = jnp.zeros_like(ss_ref)

        s_ref[...] += jnp.sum(z, axis=0, keepdims=True)
        ss_ref[...] += jnp.sum(z * z, axis=0, keepdims=True)

    in_specs = []
    args = []
    for c in range(NCH):
        in_specs += [
            pl.BlockSpec((_R, DC), lambda i: (i, 0)),
            pl.BlockSpec((_R, DC), lambda i: (i + off, 0)),
        ]
        args += [rs[c], rs[c]]
    in_specs += [pl.BlockSpec((_R, DC), lambda i: (i, 0))] * NCH
    args += list(qs)
    in_specs += [
        pl.BlockSpec((_R, 1), lambda i: (i, 0)),
        pl.BlockSpec((1, F), lambda i: (0, 0)),
    ]
    args += [dinv, bias]
    return pl.pallas_call(
        body,
        grid=(N // _R,),
        in_specs=in_specs,
        out_specs=[
            pl.BlockSpec((_R, F), lambda i: (i, 0)),
            pl.BlockSpec((1, F), lambda i: (0, 0)),
            pl.BlockSpec((1, F), lambda i: (0, 0)),
        ],
        out_shape=[
            jax.ShapeDtypeStruct((N, F), jnp.float32),
            jax.ShapeDtypeStruct((1, F), jnp.float32),
            jax.ShapeDtypeStruct((1, F), jnp.float32),
        ],
    )(*args)


def _tc_mlp(z, W, u, a=None, c=None, relu_in=True, relu_out=False,
            stats=True, dout=F):
    pre_bn = a is not None

    def body(*refs):
        if pre_bn:
            z_ref, w_ref, u_ref, a_ref, c_ref = refs[:5]
            rest = refs[5:]
            w_in = z_ref[...] * a_ref[...] + c_ref[...]
            if relu_in:
                w_in = jnp.maximum(w_in, 0.0)
        else:
            z_ref, w_ref, u_ref = refs[:3]
            rest = refs[3:]
            w_in = z_ref[...]
        v = jnp.dot(w_in, w_ref[...],
                    preferred_element_type=jnp.float32) + u_ref[...]
        if relu_out:
            v = jnp.maximum(v, 0.0)
        if stats:
            v_ref, s_ref, ss_ref = rest
            v_ref[...] = v
            i = pl.program_id(0)

            @pl.when(i == 0)
            def _():
                s_ref[...] = jnp.zeros_like(s_ref)
                ss_ref[...] = jnp.zeros_like(ss_ref)

            s_ref[...] += jnp.sum(v, axis=0, keepdims=True)
            ss_ref[...] += jnp.sum(v * v, axis=0, keepdims=True)
        else:
            rest[0][...] = v

    in_specs = [
        pl.BlockSpec((_R, F), lambda i: (i, 0)),
        pl.BlockSpec((F, dout), lambda i: (0, 0)),
        pl.BlockSpec((1, dout), lambda i: (0, 0)),
    ]
    args = [z, W, u]
    if pre_bn:
        in_specs += [pl.BlockSpec((1, F), lambda i: (0, 0))] * 2
        args += [a, c]
    out_specs = [pl.BlockSpec((_R, dout), lambda i: (i, 0))]
    out_shape = [jax.ShapeDtypeStruct((N, dout), jnp.float32)]
    if stats:
        out_specs += [pl.BlockSpec((1, dout), lambda i: (0, 0))] * 2
        out_shape += [jax.ShapeDtypeStruct((1, dout), jnp.float32)] * 2
    res = pl.pallas_call(
        body,
        grid=(N // _R,),
        in_specs=in_specs,
        out_specs=out_specs,
        out_shape=out_shape,
    )(*args)
    return res if stats else res[0]


def _padw(W, rpad=F):
    din, dout = W.shape
    return jnp.pad(W, ((0, rpad - din), (0, F - dout)))


def _padv(v, n=F):
    return jnp.pad(v, (0, n - v.shape[0])).reshape(1, n)


def _fold(s, ss, g, be):
    mu = s[0, :100] / N
    var = ss[0, :100] / N - mu * mu
    a = g * lax.rsqrt(var + 1e-5)
    c = be - mu * a
    return a, c


def _gcn_layer(h, Wp, bias, dinv, src2, dst2, a=None, c=None):
    qs = _tc_pre(h, Wp, dinv, a=a, c=c)
    rs = [_sc_aggregate(qs[k], src2, dst2) for k in range(NCH)]
    return _tc_post(rs, qs, dinv, bias)


def kernel(x, edge_index, edge_attr, Wc1, bc1, gc1, bec1, Wc2, bc2, gc2,
           bec2, Wc3, bc3, gc3, bec3, Wf1, bf1, Wf2, bf2, Wf3, bf3, gf1,
           bef1, gf2, bef2):
    src2 = edge_index[0].reshape(E // EB, EB)
    dst1 = edge_index[1]
    dst2 = dst1.reshape(E // EB, EB)

    degp = _sc_aggregate(jnp.ones((N, DC), jnp.float32), src2, dst2,
                         const_rows=True)
    dinv = _tc_dinv(degp)

    z, s, ss = _gcn_layer(x, _padw(Wc1, 75), _padv(bc1), dinv, src2, dst2)
    a, c = _fold(s, ss, gc1, bec1)
    z, s, ss = _gcn_layer(z, _padw(Wc2), _padv(bc2), dinv, src2, dst2,
                          a=_padv(a), c=_padv(c))
    a, c = _fold(s, ss, gc2, bec2)
    z, s, ss = _gcn_layer(z, _padw(Wc3), _padv(bc3), dinv, src2, dst2,
                          a=_padv(a), c=_padv(c))
    a, c = _fold(s, ss, gc3, bec3)

    v4, s4, ss4 = _tc_mlp(z, _padw(Wf1), _padv(bf1),
                          a=_padv(a), c=_padv(c), relu_in=False)
    a4, c4 = _fold(s4, ss4, gf1, bef1)
    v5, s5, ss5 = _tc_mlp(v4, _padw(Wf2), _padv(bf2),
                          a=_padv(a4), c=_padv(c4))
    a5, c5 = _fold(s5, ss5, gf2, bef2)
    out = _tc_mlp(v5, _padw(Wf3), _padv(bf3, F), a=_padv(a5), c=_padv(c5),
                  relu_out=True, stats=False)
    return out[:, :3]
